# baseline (speedup 1.0000x reference)
import functools

import jax
import jax.numpy as jnp
from jax import lax
from jax.experimental import pallas as pl
from jax.experimental.pallas import tpu as pltpu
from jax.experimental.pallas import tpu_sc as plsc

N_NODES = 10000
N_EDGES = 320000
D = 128
H = 64
NS = 16
EPT = N_EDGES // NS
CH = 10000
NST = EPT // CH
K = 80
NBLK = CH // K
NPAIR = (NBLK - 1) // 2
NCHUNK = 624
NTAIL = N_NODES - NCHUNK * NS

_SC_PARAMS = dict(
    compiler_params=pltpu.CompilerParams(needs_layout_passes=False,
                                         use_tc_tiling_on_sc=False),
)


def _i0():
    return jnp.int32(0)


def _mesh():
    return plsc.VectorSubcoreMesh(core_axis_name="c", subcore_axis_name="s")


def _zero_accs(zz_ref, accs, s):
    nb = s * jnp.int32(NCHUNK)
    for acc in accs:
        pltpu.sync_copy(zz_ref.at[pl.ds(_i0(), NCHUNK)],
                        acc.at[pl.ds(nb, NCHUNK)])

    @pl.when(s == 0)
    def _zero_tail():
        for acc in accs:
            pltpu.sync_copy(zz_ref.at[pl.ds(_i0(), NTAIL)],
                            acc.at[pl.ds(jnp.int32(NCHUNK * NS), NTAIL)])


def _write_accs(out_slices, accs, s):
    nb = s * jnp.int32(NCHUNK)
    for out_sl, acc in zip(out_slices, accs):
        pltpu.sync_copy(acc.at[pl.ds(nb, NCHUNK)], out_sl(nb, NCHUNK))

    @pl.when(s == 0)
    def _write_tail():
        tb = jnp.int32(NCHUNK * NS)
        for out_sl, acc in zip(out_slices, accs):
            pltpu.sync_copy(acc.at[pl.ds(tb, NTAIL)], out_sl(tb, NTAIL))


def _copy_idx16(dst_ref, src_ref, src_off, scale2=None):
    for i in range(K // 16):
        sl = pl.ds(src_off + i * 16, 16)
        v = src_ref[sl]
        if scale2 is not None:
            v = v * 2 + scale2
        dst_ref[pl.ds(i * 16, 16)] = v


def _square_rows(dst_ref, src_ref):
    def body(e2, carry):
        e = e2 * jnp.int32(2)
        for r in range(2):
            for ci in range(H // 16):
                sl = pl.ds(jnp.int32(ci * 16), 16)
                v = src_ref[e + r, sl]
                dst_ref[e + r, sl] = v * v
        return carry

    lax.fori_loop(_i0(), jnp.int32(K // 2), body, _i0())


def _scale_rows(dst_ref, src_ref, val_ref, vbase):
    def body(g, carry):
        e0 = g * jnp.int32(16)
        vv = val_ref[pl.ds(vbase + e0, 16)]
        for r in range(16):
            bv = jnp.full((16,), vv[r], jnp.float32)
            e = e0 + r
            for ci in range(H // 16):
                sl = pl.ds(jnp.int32(ci * 16), 16)
                dst_ref[e, sl] = src_ref[e, sl] * bv
        return carry

    lax.fori_loop(_i0(), jnp.int32(K // 16), body, _i0())


def _sc_segments(x_r, src_h, dst_h, val_h, zz):
    @functools.partial(
        pl.kernel,
        mesh=_mesh(),
        out_type=(jax.ShapeDtypeStruct((2, 2, N_NODES, H), jnp.float32),
                  jax.ShapeDtypeStruct((2, N_NODES, H), jnp.float32)),
        scratch_types=[
            pltpu.VMEM((CH,), jnp.int32),
            pltpu.VMEM((CH,), jnp.int32),
            pltpu.VMEM((CH + 16,), jnp.float32),
            pltpu.VMEM((K,), jnp.int32),
            pltpu.VMEM((K,), jnp.int32),
            pltpu.VMEM((K,), jnp.int32),
            pltpu.VMEM((K,), jnp.int32),
            pltpu.VMEM((K, H), jnp.float32),
            pltpu.VMEM((K, H), jnp.float32),
            pltpu.VMEM((K, H), jnp.float32),
            pltpu.VMEM_SHARED((N_NODES, H), jnp.float32),
            pltpu.VMEM_SHARED((N_NODES, H), jnp.float32),
            pltpu.SemaphoreType.DMA,
            pltpu.SemaphoreType.DMA,
        ],
        **_SC_PARAMS,
    )
    def sc_fn(x_ref, src_ref, dst_ref, val_ref, zz_ref, out_sp, out_g,
              srcS, dstS, valS, srcv0, srcv1, dstv0, dstv1,
              rows0, rows1, tmpb, accs, accp, semg0, semg1):
        c = lax.axis_index("c")
        s = lax.axis_index("s")
        _zero_accs(zz_ref, (accs, accp), s)
        plsc.subcore_barrier()

        srcv = (srcv0, srcv1)
        dstv = (dstv0, dstv1)
        rows = (rows0, rows1)
        semg = (semg0, semg1)
        ebase = s * jnp.int32(EPT)

        def prefetch(q, base):
            _copy_idx16(srcv[q], srcS, base, scale2=c)
            _copy_idx16(dstv[q], dstS, base)
            pltpu.async_copy(x_ref.at[srcv[q]], rows[q], semg[q])

        def run_pass(stage_extra, process):
            for t in range(NST):
                sb = ebase + jnp.int32(t * CH)
                pltpu.sync_copy(src_ref.at[pl.ds(sb, CH)], srcS)
                pltpu.sync_copy(dst_ref.at[pl.ds(sb, CH)], dstS)
                stage_extra(sb)
                prefetch(0, _i0())

                def pair_body(jp, carry):
                    j2 = jp * jnp.int32(2 * K)
                    for p in range(2):
                        prefetch(1 - p, j2 + jnp.int32((p + 1) * K))
                        process(p, j2 + jnp.int32(p * K))
                    return carry

                lax.fori_loop(_i0(), jnp.int32(NPAIR), pair_body, _i0())
                process(0, jnp.int32((NBLK - 1) * K))

        def process1(p, base):
            pltpu.make_async_copy(x_ref.at[srcv[p]], rows[p], semg[p]).wait()
            _square_rows(tmpb, rows[p])
            pltpu.sync_copy(rows[p], accs.at[dstv[p]], add=True)
            pltpu.sync_copy(tmpb, accp.at[dstv[p]], add=True)

        run_pass(lambda sb: None, process1)
        plsc.subcore_barrier()
        _write_accs(
            (lambda nb, nn: out_sp.at[c, _i0(), pl.ds(nb, nn)],
             lambda nb, nn: out_sp.at[c, jnp.int32(1), pl.ds(nb, nn)]),
            (accs, accp), s)
        _zero_accs(zz_ref, (accs,), s)
        plsc.subcore_barrier()

        def stage_val(sb):
            pltpu.sync_copy(val_ref.at[pl.ds(sb, CH)],
                            valS.at[pl.ds(_i0(), CH)])

        def process2(p, base):
            pltpu.make_async_copy(x_ref.at[srcv[p]], rows[p], semg[p]).wait()
            _scale_rows(tmpb, rows[p], valS, base)
            pltpu.sync_copy(tmpb, accs.at[dstv[p]], add=True)

        run_pass(stage_val, process2)
        plsc.subcore_barrier()
        _write_accs((lambda nb, nn: out_g.at[c, pl.ds(nb, nn)],),
                    (accs,), s)

    return sc_fn(x_r, src_h, dst_h, val_h, zz)


def _epilogue_kernel(g0, g1, s0, s1, p0, p1, diag, Wg0, Wg1, Wp0, Wp1, b):
    bn = 400

    def body(g0_r, g1_r, s0_r, s1_r, p0_r, p1_r, d_r, wg0_r, wg1_r,
             wp0_r, wp1_r, b_r, o_r):
        d = d_r[...]
        pna0 = 0.5 * d * (s0_r[...] * s0_r[...] - p0_r[...])
        pna1 = 0.5 * d * (s1_r[...] * s1_r[...] - p1_r[...])
        h = jnp.dot(g0_r[...], wg0_r[...], preferred_element_type=jnp.float32)
        h += jnp.dot(g1_r[...], wg1_r[...], preferred_element_type=jnp.float32)
        h += jnp.dot(pna0, wp0_r[...], preferred_element_type=jnp.float32)
        h += jnp.dot(pna1, wp1_r[...], preferred_element_type=jnp.float32)
        h += b_r[...]
        o_r[...] = jnp.where(h > 0, h, 0.2 * h)

    half = pl.BlockSpec((bn, H), lambda i: (i, _i0()))
    wspec = pl.BlockSpec((H, D), lambda i: (_i0(), _i0()))
    return pl.pallas_call(
        body,
        grid=(N_NODES // bn,),
        in_specs=[half, half, half, half, half, half,
                  pl.BlockSpec((bn, 1), lambda i: (i, _i0())),
                  wspec, wspec, wspec, wspec,
                  pl.BlockSpec((1, D), lambda i: (_i0(), _i0()))],
        out_specs=pl.BlockSpec((bn, D), lambda i: (i, _i0())),
        out_shape=jax.ShapeDtypeStruct((N_NODES, D), jnp.float32),
    )(g0, g1, s0, s1, p0, p1, diag, Wg0, Wg1, Wp0, Wp1, b)


def kernel(users_emb, items_emb, edge_index, graph_vals, diag_vals, W, b):
    num_user = users_emb.shape[0]
    x = jnp.concatenate([users_emb, items_emb], axis=0)
    x_r = x.reshape(2 * N_NODES, H)
    dst32 = edge_index[0].astype(jnp.int32)
    src32 = edge_index[1].astype(jnp.int32)
    val32 = graph_vals.astype(jnp.float32)
    zz = jnp.zeros((NCHUNK, H), jnp.float32)

    osp, og = _sc_segments(x_r, src32, dst32, val32, zz)

    diag = diag_vals.astype(jnp.float32).reshape(N_NODES, 1)
    Wf = W.astype(jnp.float32)
    Wg0, Wg1 = Wf[:H], Wf[H:D]
    Wp0, Wp1 = Wf[D:D + H], Wf[D + H:]
    b2 = b.astype(jnp.float32).reshape(1, D)

    out = _epilogue_kernel(og[0], og[1], osp[0, 0], osp[1, 0], osp[0, 1],
                           osp[1, 1], diag, Wg0, Wg1, Wp0, Wp1, b2)
    out64 = out.astype(jnp.float64)
    return (out64[:num_user], out64[num_user:])


# speedup vs baseline: 1.0118x; 1.0118x over previous
import functools

import jax
import jax.numpy as jnp
from jax import lax
from jax.experimental import pallas as pl
from jax.experimental.pallas import tpu as pltpu
from jax.experimental.pallas import tpu_sc as plsc

N_NODES = 10000
N_EDGES = 320000
D = 128
H = 64
NS = 16
EPT = N_EDGES // NS
CH = 10000
NST = EPT // CH
K = 80
NBLK = CH // K
NPAIR = (NBLK - 1) // 2
NCHUNK = 624
NTAIL = N_NODES - NCHUNK * NS

_SC_PARAMS = dict(
    compiler_params=pltpu.CompilerParams(needs_layout_passes=False,
                                         use_tc_tiling_on_sc=False),
)


def _i0():
    return jnp.int32(0)


def _mesh():
    return plsc.VectorSubcoreMesh(core_axis_name="c", subcore_axis_name="s")


def _zero_accs(zz_ref, accs, s):
    nb = s * jnp.int32(NCHUNK)
    for acc in accs:
        pltpu.sync_copy(zz_ref.at[pl.ds(_i0(), NCHUNK)],
                        acc.at[pl.ds(nb, NCHUNK)])

    @pl.when(s == 0)
    def _zero_tail():
        for acc in accs:
            pltpu.sync_copy(zz_ref.at[pl.ds(_i0(), NTAIL)],
                            acc.at[pl.ds(jnp.int32(NCHUNK * NS), NTAIL)])


def _write_accs(out_slices, accs, s):
    nb = s * jnp.int32(NCHUNK)
    for out_sl, acc in zip(out_slices, accs):
        pltpu.sync_copy(acc.at[pl.ds(nb, NCHUNK)], out_sl(nb, NCHUNK))

    @pl.when(s == 0)
    def _write_tail():
        tb = jnp.int32(NCHUNK * NS)
        for out_sl, acc in zip(out_slices, accs):
            pltpu.sync_copy(acc.at[pl.ds(tb, NTAIL)], out_sl(tb, NTAIL))


def _copy_idx16(dst_ref, src_ref, src_off, scale2=None):
    for i in range(K // 16):
        sl = pl.ds(src_off + i * 16, 16)
        v = src_ref[sl]
        if scale2 is not None:
            v = v * 2 + scale2
        dst_ref[pl.ds(i * 16, 16)] = v


def _square_rows(dst_ref, src_ref):
    def body(e8, carry):
        e = e8 * jnp.int32(8)
        for r in range(8):
            for ci in range(H // 16):
                sl = pl.ds(jnp.int32(ci * 16), 16)
                v = src_ref[e + r, sl]
                dst_ref[e + r, sl] = v * v
        return carry

    lax.fori_loop(_i0(), jnp.int32(K // 8), body, _i0())


def _scale_rows(dst_ref, src_ref, val_ref, vbase):
    def body(g, carry):
        e0 = g * jnp.int32(16)
        vv = val_ref[pl.ds(vbase + e0, 16)]
        for r in range(16):
            bv = jnp.full((16,), vv[r], jnp.float32)
            e = e0 + r
            for ci in range(H // 16):
                sl = pl.ds(jnp.int32(ci * 16), 16)
                dst_ref[e, sl] = src_ref[e, sl] * bv
        return carry

    lax.fori_loop(_i0(), jnp.int32(K // 16), body, _i0())


def _sc_segments(x_r, src_h, dst_h, val_h, zz):
    @functools.partial(
        pl.kernel,
        mesh=_mesh(),
        out_type=(jax.ShapeDtypeStruct((2, 2, N_NODES, H), jnp.float32),
                  jax.ShapeDtypeStruct((2, N_NODES, H), jnp.float32)),
        scratch_types=[
            pltpu.VMEM((CH,), jnp.int32),
            pltpu.VMEM((CH,), jnp.int32),
            pltpu.VMEM((CH + 16,), jnp.float32),
            pltpu.VMEM((K,), jnp.int32),
            pltpu.VMEM((K,), jnp.int32),
            pltpu.VMEM((K,), jnp.int32),
            pltpu.VMEM((K,), jnp.int32),
            pltpu.VMEM((K, H), jnp.float32),
            pltpu.VMEM((K, H), jnp.float32),
            pltpu.VMEM((K, H), jnp.float32),
            pltpu.VMEM_SHARED((N_NODES, H), jnp.float32),
            pltpu.VMEM_SHARED((N_NODES, H), jnp.float32),
            pltpu.SemaphoreType.DMA,
            pltpu.SemaphoreType.DMA,
            pltpu.SemaphoreType.DMA,
            pltpu.SemaphoreType.DMA,
        ],
        **_SC_PARAMS,
    )
    def sc_fn(x_ref, src_ref, dst_ref, val_ref, zz_ref, out_sp, out_g,
              srcS, dstS, valS, srcv0, srcv1, dstv0, dstv1,
              rows0, rows1, tmpb, accs, accp, semg0, semg1, sema, semb):
        c = lax.axis_index("c")
        s = lax.axis_index("s")
        _zero_accs(zz_ref, (accs, accp), s)
        plsc.subcore_barrier()

        srcv = (srcv0, srcv1)
        dstv = (dstv0, dstv1)
        rows = (rows0, rows1)
        semg = (semg0, semg1)
        ebase = s * jnp.int32(EPT)

        def prefetch(q, base):
            _copy_idx16(srcv[q], srcS, base, scale2=c)
            _copy_idx16(dstv[q], dstS, base)
            pltpu.async_copy(x_ref.at[srcv[q]], rows[q], semg[q])

        def run_pass(stage_extra, process):
            for t in range(NST):
                sb = ebase + jnp.int32(t * CH)
                pltpu.sync_copy(src_ref.at[pl.ds(sb, CH)], srcS)
                pltpu.sync_copy(dst_ref.at[pl.ds(sb, CH)], dstS)
                stage_extra(sb)
                prefetch(0, _i0())

                def pair_body(jp, carry):
                    j2 = jp * jnp.int32(2 * K)
                    for p in range(2):
                        prefetch(1 - p, j2 + jnp.int32((p + 1) * K))
                        process(p, j2 + jnp.int32(p * K))
                    return carry

                lax.fori_loop(_i0(), jnp.int32(NPAIR), pair_body, _i0())
                process(0, jnp.int32((NBLK - 1) * K))

        def process1(p, base):
            pltpu.make_async_copy(x_ref.at[srcv[p]], rows[p], semg[p]).wait()
            _square_rows(tmpb, rows[p])
            cpa = pltpu.async_copy(rows[p], accs.at[dstv[p]], sema, add=True)
            cpb = pltpu.async_copy(tmpb, accp.at[dstv[p]], semb, add=True)
            cpa.wait()
            cpb.wait()

        run_pass(lambda sb: None, process1)
        plsc.subcore_barrier()
        _write_accs(
            (lambda nb, nn: out_sp.at[c, _i0(), pl.ds(nb, nn)],
             lambda nb, nn: out_sp.at[c, jnp.int32(1), pl.ds(nb, nn)]),
            (accs, accp), s)
        _zero_accs(zz_ref, (accs,), s)
        plsc.subcore_barrier()

        def stage_val(sb):
            pltpu.sync_copy(val_ref.at[pl.ds(sb, CH)],
                            valS.at[pl.ds(_i0(), CH)])

        def process2(p, base):
            pltpu.make_async_copy(x_ref.at[srcv[p]], rows[p], semg[p]).wait()
            _scale_rows(tmpb, rows[p], valS, base)
            pltpu.sync_copy(tmpb, accs.at[dstv[p]], add=True)

        run_pass(stage_val, process2)
        plsc.subcore_barrier()
        _write_accs((lambda nb, nn: out_g.at[c, pl.ds(nb, nn)],),
                    (accs,), s)

    return sc_fn(x_r, src_h, dst_h, val_h, zz)


def _epilogue_kernel(g0, g1, s0, s1, p0, p1, diag, Wg0, Wg1, Wp0, Wp1, b):
    bn = 400

    def body(g0_r, g1_r, s0_r, s1_r, p0_r, p1_r, d_r, wg0_r, wg1_r,
             wp0_r, wp1_r, b_r, o_r):
        d = d_r[...]
        pna0 = 0.5 * d * (s0_r[...] * s0_r[...] - p0_r[...])
        pna1 = 0.5 * d * (s1_r[...] * s1_r[...] - p1_r[...])
        h = jnp.dot(g0_r[...], wg0_r[...], preferred_element_type=jnp.float32)
        h += jnp.dot(g1_r[...], wg1_r[...], preferred_element_type=jnp.float32)
        h += jnp.dot(pna0, wp0_r[...], preferred_element_type=jnp.float32)
        h += jnp.dot(pna1, wp1_r[...], preferred_element_type=jnp.float32)
        h += b_r[...]
        o_r[...] = jnp.where(h > 0, h, 0.2 * h)

    half = pl.BlockSpec((bn, H), lambda i: (i, _i0()))
    wspec = pl.BlockSpec((H, D), lambda i: (_i0(), _i0()))
    return pl.pallas_call(
        body,
        grid=(N_NODES // bn,),
        in_specs=[half, half, half, half, half, half,
                  pl.BlockSpec((bn, 1), lambda i: (i, _i0())),
                  wspec, wspec, wspec, wspec,
                  pl.BlockSpec((1, D), lambda i: (_i0(), _i0()))],
        out_specs=pl.BlockSpec((bn, D), lambda i: (i, _i0())),
        out_shape=jax.ShapeDtypeStruct((N_NODES, D), jnp.float32),
    )(g0, g1, s0, s1, p0, p1, diag, Wg0, Wg1, Wp0, Wp1, b)


def kernel(users_emb, items_emb, edge_index, graph_vals, diag_vals, W, b):
    num_user = users_emb.shape[0]
    x = jnp.concatenate([users_emb, items_emb], axis=0)
    x_r = x.reshape(2 * N_NODES, H)
    dst32 = edge_index[0].astype(jnp.int32)
    src32 = edge_index[1].astype(jnp.int32)
    val32 = graph_vals.astype(jnp.float32)
    zz = jnp.zeros((NCHUNK, H), jnp.float32)

    osp, og = _sc_segments(x_r, src32, dst32, val32, zz)

    diag = diag_vals.astype(jnp.float32).reshape(N_NODES, 1)
    Wf = W.astype(jnp.float32)
    Wg0, Wg1 = Wf[:H], Wf[H:D]
    Wp0, Wp1 = Wf[D:D + H], Wf[D + H:]
    b2 = b.astype(jnp.float32).reshape(1, D)

    out = _epilogue_kernel(og[0], og[1], osp[0, 0], osp[1, 0], osp[0, 1],
                           osp[1, 1], diag, Wg0, Wg1, Wp0, Wp1, b2)
    out64 = out.astype(jnp.float64)
    return (out64[:num_user], out64[num_user:])
